# Initial kernel scaffold; baseline (speedup 1.0000x reference)
import jax
import jax.numpy as jnp
from jax.experimental import pallas as pl


def kernel(x, edge_index, W1, b1, W2, b2):
    raise NotImplementedError("write your pallas kernel here")


# speedup vs baseline: 18.1847x; 18.1847x over previous
import functools

import jax
import jax.numpy as jnp
from jax import lax
from jax.experimental import pallas as pl
from jax.experimental.pallas import tpu as pltpu
from jax.experimental.pallas import tpu_sc as plsc

N = 10000
E = 320000
K = 80
NC, NS = 2, 16
NP = 10240
RPT = NP // NS
CHUNK = 2000
NBC = CHUNK // K

_sc_mesh = plsc.VectorSubcoreMesh(core_axis_name="c", subcore_axis_name="s")


@functools.partial(
    pl.kernel,
    out_type=jax.ShapeDtypeStruct((NC, NP, 128), jnp.float32),
    mesh=_sc_mesh,
    scratch_types=[
        pltpu.VMEM_SHARED((NP, 128), jnp.float32),
        pltpu.VMEM((E // (NC * NS) // K, K), jnp.int32),
        pltpu.VMEM((K, 128), jnp.float32),
    ],
)
def _deg_kernel(dst_hbm, ones_hbm, zeros_hbm, out_hbm, accum, dstbuf, onesbuf):
    c = lax.axis_index("c")
    s = lax.axis_index("s")
    nb = E // (NC * NS) // K
    pltpu.sync_copy(zeros_hbm, accum.at[pl.ds(s * RPT, RPT)])
    pltpu.sync_copy(ones_hbm, onesbuf)
    pltpu.sync_copy(dst_hbm.at[c * NS + s], dstbuf)
    plsc.subcore_barrier()

    def body(j, carry):
        pltpu.sync_copy(onesbuf, accum.at[dstbuf.at[j]], add=True)
        return carry

    lax.fori_loop(0, nb, body, 0)
    plsc.subcore_barrier()
    pltpu.sync_copy(accum.at[pl.ds(s * RPT, RPT)],
                    out_hbm.at[c, pl.ds(s * RPT, RPT)])


def _make_agg(src_rows, split_edges):
    ept = E // (NC * NS) if split_edges else E // NS
    nch = ept // CHUNK

    @functools.partial(
        pl.kernel,
        out_type=jax.ShapeDtypeStruct((NC, NP, 128), jnp.float32),
        mesh=_sc_mesh,
        scratch_types=[
            pltpu.VMEM_SHARED((NP, 128), jnp.float32),
            pltpu.VMEM((CHUNK,), jnp.int32),
            pltpu.VMEM((NBC, K), jnp.int32),
            pltpu.VMEM((K, 128), jnp.float32),
        ],
    )
    def agg(feat_hbm, src_hbm, dst_hbm, zeros_hbm, out_hbm,
            accum, srcbuf, dstbuf, rows):
        c = lax.axis_index("c")
        s = lax.axis_index("s")
        pltpu.sync_copy(zeros_hbm, accum.at[pl.ds(s * RPT, RPT)])
        if split_edges:
            tile = c * NS + s
        else:
            tile = s
        plsc.subcore_barrier()

        def chunk_body(q, carry):
            pltpu.sync_copy(src_hbm.at[pl.ds(tile * ept + q * CHUNK, CHUNK)],
                            srcbuf)
            pltpu.sync_copy(dst_hbm.at[tile, q], dstbuf)
            if not split_edges:
                off = c * N

                def addoff(i, carry2):
                    srcbuf[pl.ds(i * 16, 16)] = srcbuf[pl.ds(i * 16, 16)] + off
                    return carry2

                lax.fori_loop(0, CHUNK // 16, addoff, 0)

            def body(j, carry2):
                pltpu.sync_copy(feat_hbm.at[srcbuf.at[pl.ds(j * K, K)]], rows)
                pltpu.sync_copy(rows, accum.at[dstbuf.at[j]], add=True)
                return carry2

            lax.fori_loop(0, NBC, body, 0)
            return carry

        lax.fori_loop(0, nch, chunk_body, 0)
        plsc.subcore_barrier()
        pltpu.sync_copy(accum.at[pl.ds(s * RPT, RPT)],
                        out_hbm.at[c, pl.ds(s * RPT, RPT)])

    return agg


_agg_split = _make_agg(N, True)
_agg_feat = _make_agg(2 * N, False)


_BLK = 1000


def _prescale_call(degp, x):
    def body(degp_ref, x_ref, xs_ref, dis_ref):
        deg = degp_ref[0, :, 0:1] + degp_ref[1, :, 0:1] + 1.0
        d = lax.rsqrt(deg)
        dis_ref[...] = d
        xs_ref[...] = x_ref[...] * d

    return pl.pallas_call(
        body,
        grid=(N // _BLK,),
        in_specs=[
            pl.BlockSpec((NC, _BLK, 128), lambda i: (0, i, 0)),
            pl.BlockSpec((_BLK, 128), lambda i: (i, 0)),
        ],
        out_specs=[
            pl.BlockSpec((_BLK, 128), lambda i: (i, 0)),
            pl.BlockSpec((_BLK, 1), lambda i: (i, 0)),
        ],
        out_shape=[
            jax.ShapeDtypeStruct((N, 128), jnp.float32),
            jax.ShapeDtypeStruct((N, 1), jnp.float32),
        ],
    )(degp, x)


def _layer1_call(s1, x, dis, W1, b1, W2p):
    def body(s1_ref, x_ref, dis_ref, W1_ref, b1_ref, W2_ref, out_ref):
        d = dis_ref[...]
        agg = d * (s1_ref[0] + s1_ref[1]) + (d * d) * x_ref[...]
        h1 = jnp.maximum(
            jnp.dot(agg, W1_ref[...], preferred_element_type=jnp.float32)
            + b1_ref[...], 0.0)
        g = jnp.dot(h1, W2_ref[...], preferred_element_type=jnp.float32)
        out_ref[...] = d * g

    return pl.pallas_call(
        body,
        grid=(N // _BLK, 2),
        in_specs=[
            pl.BlockSpec((NC, _BLK, 128), lambda i, j: (0, i, 0)),
            pl.BlockSpec((_BLK, 128), lambda i, j: (i, 0)),
            pl.BlockSpec((_BLK, 1), lambda i, j: (i, 0)),
            pl.BlockSpec((128, 512), lambda i, j: (0, 0)),
            pl.BlockSpec((1, 512), lambda i, j: (0, 0)),
            pl.BlockSpec((512, 128), lambda i, j: (0, j)),
        ],
        out_specs=pl.BlockSpec((_BLK, 128),
                               lambda i, j: (j * (N // _BLK) + i, 0)),
        out_shape=jax.ShapeDtypeStruct((2 * N, 128), jnp.float32),
    )(s1, x, dis, W1, b1, W2p)


def _layer2_call(s2, gsc, dis, b2p):
    def body(s2_ref, gsc_ref, dis_ref, b2_ref, out_ref):
        d = dis_ref[...]
        b = jnp.where(pl.program_id(1) == 0, b2_ref[0:1, :], b2_ref[1:2, :])
        out_ref[...] = jnp.maximum(
            d * (s2_ref[0] + gsc_ref[...]) + b, 0.0)

    return pl.pallas_call(
        body,
        grid=(N // _BLK, 2),
        in_specs=[
            pl.BlockSpec((1, _BLK, 128), lambda i, j: (j, i, 0)),
            pl.BlockSpec((_BLK, 128), lambda i, j: (j * (N // _BLK) + i, 0)),
            pl.BlockSpec((_BLK, 1), lambda i, j: (i, 0)),
            pl.BlockSpec((2, 128), lambda i, j: (0, 0)),
        ],
        out_specs=pl.BlockSpec((_BLK, 128), lambda i, j: (i, j)),
        out_shape=jax.ShapeDtypeStruct((N, 256), jnp.float32),
    )(s2, gsc, dis, b2p)


def kernel(x, edge_index, W1, b1, W2, b2):
    src = edge_index[0]
    dst3s = edge_index[1].reshape(NC * NS, E // (NC * NS) // CHUNK, NBC, K)
    dst3f = edge_index[1].reshape(NS, E // NS // CHUNK, NBC, K)
    dst3d = edge_index[1].reshape(NC * NS, E // (NC * NS) // K, K)
    ones128 = jnp.ones((K, 128), jnp.float32)
    zeros128 = jnp.zeros((RPT, 128), jnp.float32)
    W2p = jnp.pad(W2, ((0, 0), (0, 256 - W2.shape[1])))
    b2p = jnp.pad(b2, (0, 256 - b2.shape[0])).reshape(2, 128)

    degp = _deg_kernel(dst3d, ones128, zeros128)
    xs, dis = _prescale_call(degp, x)
    s1 = _agg_split(xs, src, dst3s, zeros128)
    gsc = _layer1_call(s1, x, dis, W1, b1.reshape(1, 512), W2p)
    s2 = _agg_feat(gsc, src, dst3f, zeros128)
    o = _layer2_call(s2, gsc, dis, b2p)
    return o[:, :250]


# speedup vs baseline: 22.1858x; 1.2200x over previous
import functools

import jax
import jax.numpy as jnp
from jax import lax
from jax.experimental import pallas as pl
from jax.experimental.pallas import tpu as pltpu
from jax.experimental.pallas import tpu_sc as plsc

N = 10000
E = 320000
K = 80
NC, NS = 2, 16
NP = 10240
RPT = NP // NS
CHUNK = 2000
NBC = CHUNK // K

_sc_mesh = plsc.VectorSubcoreMesh(core_axis_name="c", subcore_axis_name="s")


@functools.partial(
    pl.kernel,
    out_type=jax.ShapeDtypeStruct((NC, NP, 128), jnp.float32),
    mesh=_sc_mesh,
    scratch_types=[
        pltpu.VMEM_SHARED((NP, 128), jnp.float32),
        pltpu.VMEM((E // (NC * NS) // K, K), jnp.int32),
        pltpu.VMEM((K, 128), jnp.float32),
    ],
)
def _deg_kernel(dst_hbm, ones_hbm, zeros_hbm, out_hbm, accum, dstbuf, onesbuf):
    c = lax.axis_index("c")
    s = lax.axis_index("s")
    nb = E // (NC * NS) // K
    pltpu.sync_copy(zeros_hbm, accum.at[pl.ds(s * RPT, RPT)])
    pltpu.sync_copy(ones_hbm, onesbuf)
    pltpu.sync_copy(dst_hbm.at[c * NS + s], dstbuf)
    plsc.subcore_barrier()

    def body(j, carry):
        pltpu.sync_copy(onesbuf, accum.at[dstbuf.at[j]], add=True)
        return carry

    lax.fori_loop(0, nb, body, 0)
    plsc.subcore_barrier()
    pltpu.sync_copy(accum.at[pl.ds(s * RPT, RPT)],
                    out_hbm.at[c, pl.ds(s * RPT, RPT)])


def _make_agg(src_rows, split_edges):
    ept = E // (NC * NS) if split_edges else E // NS
    nch = ept // CHUNK

    @functools.partial(
        pl.kernel,
        out_type=jax.ShapeDtypeStruct((NC, NP, 128), jnp.float32),
        mesh=_sc_mesh,
        scratch_types=[
            pltpu.VMEM_SHARED((NP, 128), jnp.float32),
            pltpu.VMEM((CHUNK,), jnp.int32),
            pltpu.VMEM((NBC, K), jnp.int32),
            pltpu.VMEM((K, 128), jnp.float32),
            pltpu.VMEM((K, 128), jnp.float32),
            pltpu.SemaphoreType.DMA,
            pltpu.SemaphoreType.DMA,
        ],
    )
    def agg(feat_hbm, src_hbm, dst_hbm, zeros_hbm, out_hbm,
            accum, srcbuf, dstbuf, rows0, rows1, gsem, ssem):
        c = lax.axis_index("c")
        s = lax.axis_index("s")
        pltpu.sync_copy(zeros_hbm, accum.at[pl.ds(s * RPT, RPT)])
        if split_edges:
            tile = c * NS + s
        else:
            tile = s
        plsc.subcore_barrier()

        def wait_gather(buf):
            pltpu.make_async_copy(
                feat_hbm.at[srcbuf.at[pl.ds(0, K)]], buf, gsem).wait()

        def wait_scatter():
            pltpu.make_async_copy(
                rows0, accum.at[dstbuf.at[0]], ssem).wait()

        def chunk_body(q, carry):
            pltpu.sync_copy(src_hbm.at[pl.ds(tile * ept + q * CHUNK, CHUNK)],
                            srcbuf)
            pltpu.sync_copy(dst_hbm.at[tile, q], dstbuf)
            if not split_edges:
                off = c * N

                def addoff(i, carry2):
                    srcbuf[pl.ds(i * 16, 16)] = srcbuf[pl.ds(i * 16, 16)] + off
                    return carry2

                lax.fori_loop(0, CHUNK // 16, addoff, 0)

            pltpu.async_copy(feat_hbm.at[srcbuf.at[pl.ds(0, K)]], rows0, gsem)

            def step(j, carry2):
                def stage(cur, nxt):
                    wait_gather(cur)

                    @pl.when(j + 1 < NBC)
                    def _():
                        @pl.when(j >= 1)
                        def _():
                            wait_scatter()

                        pltpu.async_copy(
                            feat_hbm.at[srcbuf.at[pl.ds((j + 1) * K, K)]],
                            nxt, gsem)

                    pltpu.async_copy(cur, accum.at[dstbuf.at[j]], ssem,
                                     add=True)

                @pl.when(j % 2 == 0)
                def _():
                    stage(rows0, rows1)

                @pl.when(j % 2 == 1)
                def _():
                    stage(rows1, rows0)

                return carry2

            lax.fori_loop(0, NBC, step, 0)
            wait_scatter()
            wait_scatter()
            return carry

        lax.fori_loop(0, nch, chunk_body, 0)
        plsc.subcore_barrier()
        pltpu.sync_copy(accum.at[pl.ds(s * RPT, RPT)],
                        out_hbm.at[c, pl.ds(s * RPT, RPT)])

    return agg


_agg_split = _make_agg(N, True)
_agg_feat = _make_agg(2 * N, False)


_BLK = 1000


def _prescale_call(degp, x):
    def body(degp_ref, x_ref, xs_ref, dis_ref):
        deg = degp_ref[0, :, 0:1] + degp_ref[1, :, 0:1] + 1.0
        d = lax.rsqrt(deg)
        dis_ref[...] = d
        xs_ref[...] = x_ref[...] * d

    return pl.pallas_call(
        body,
        grid=(N // _BLK,),
        in_specs=[
            pl.BlockSpec((NC, _BLK, 128), lambda i: (0, i, 0)),
            pl.BlockSpec((_BLK, 128), lambda i: (i, 0)),
        ],
        out_specs=[
            pl.BlockSpec((_BLK, 128), lambda i: (i, 0)),
            pl.BlockSpec((_BLK, 1), lambda i: (i, 0)),
        ],
        out_shape=[
            jax.ShapeDtypeStruct((N, 128), jnp.float32),
            jax.ShapeDtypeStruct((N, 1), jnp.float32),
        ],
    )(degp, x)


def _layer1_call(s1, x, dis, W1, b1, W2p):
    def body(s1_ref, x_ref, dis_ref, W1_ref, b1_ref, W2_ref, out_ref):
        d = dis_ref[...]
        agg = d * (s1_ref[0] + s1_ref[1]) + (d * d) * x_ref[...]
        h1 = jnp.maximum(
            jnp.dot(agg, W1_ref[...], preferred_element_type=jnp.float32)
            + b1_ref[...], 0.0)
        g = jnp.dot(h1, W2_ref[...], preferred_element_type=jnp.float32)
        out_ref[...] = d * g

    return pl.pallas_call(
        body,
        grid=(N // _BLK, 2),
        in_specs=[
            pl.BlockSpec((NC, _BLK, 128), lambda i, j: (0, i, 0)),
            pl.BlockSpec((_BLK, 128), lambda i, j: (i, 0)),
            pl.BlockSpec((_BLK, 1), lambda i, j: (i, 0)),
            pl.BlockSpec((128, 512), lambda i, j: (0, 0)),
            pl.BlockSpec((1, 512), lambda i, j: (0, 0)),
            pl.BlockSpec((512, 128), lambda i, j: (0, j)),
        ],
        out_specs=pl.BlockSpec((_BLK, 128),
                               lambda i, j: (j * (N // _BLK) + i, 0)),
        out_shape=jax.ShapeDtypeStruct((2 * N, 128), jnp.float32),
    )(s1, x, dis, W1, b1, W2p)


def _layer2_call(s2, gsc, dis, b2p):
    def body(s2_ref, gsc_ref, dis_ref, b2_ref, out_ref):
        d = dis_ref[...]
        b = jnp.where(pl.program_id(1) == 0, b2_ref[0:1, :], b2_ref[1:2, :])
        out_ref[...] = jnp.maximum(
            d * (s2_ref[0] + gsc_ref[...]) + b, 0.0)

    return pl.pallas_call(
        body,
        grid=(N // _BLK, 2),
        in_specs=[
            pl.BlockSpec((1, _BLK, 128), lambda i, j: (j, i, 0)),
            pl.BlockSpec((_BLK, 128), lambda i, j: (j * (N // _BLK) + i, 0)),
            pl.BlockSpec((_BLK, 1), lambda i, j: (i, 0)),
            pl.BlockSpec((2, 128), lambda i, j: (0, 0)),
        ],
        out_specs=pl.BlockSpec((_BLK, 128), lambda i, j: (i, j)),
        out_shape=jax.ShapeDtypeStruct((N, 256), jnp.float32),
    )(s2, gsc, dis, b2p)


def kernel(x, edge_index, W1, b1, W2, b2):
    src = edge_index[0]
    dst3s = edge_index[1].reshape(NC * NS, E // (NC * NS) // CHUNK, NBC, K)
    dst3f = edge_index[1].reshape(NS, E // NS // CHUNK, NBC, K)
    dst3d = edge_index[1].reshape(NC * NS, E // (NC * NS) // K, K)
    ones128 = jnp.ones((K, 128), jnp.float32)
    zeros128 = jnp.zeros((RPT, 128), jnp.float32)
    W2p = jnp.pad(W2, ((0, 0), (0, 256 - W2.shape[1])))
    b2p = jnp.pad(b2, (0, 256 - b2.shape[0])).reshape(2, 128)

    degp = _deg_kernel(dst3d, ones128, zeros128)
    xs, dis = _prescale_call(degp, x)
    s1 = _agg_split(xs, src, dst3s, zeros128)
    gsc = _layer1_call(s1, x, dis, W1, b1.reshape(1, 512), W2p)
    s2 = _agg_feat(gsc, src, dst3f, zeros128)
    o = _layer2_call(s2, gsc, dis, b2p)
    return o[:, :250]


# speedup vs baseline: 29.3567x; 1.3232x over previous
import functools

import jax
import jax.numpy as jnp
from jax import lax
from jax.experimental import pallas as pl
from jax.experimental.pallas import tpu as pltpu
from jax.experimental.pallas import tpu_sc as plsc

N = 10000
E = 320000
K = 80
NC, NS = 2, 16
NP = 10240
RPT = NP // NS
CHUNK = 2000
NBC = CHUNK // K

_sc_mesh = plsc.VectorSubcoreMesh(core_axis_name="c", subcore_axis_name="s")


@functools.partial(
    pl.kernel,
    out_type=jax.ShapeDtypeStruct((NC, NP, 128), jnp.float32),
    mesh=_sc_mesh,
    scratch_types=[
        pltpu.VMEM_SHARED((NP, 128), jnp.float32),
        pltpu.VMEM((E // (NC * NS) // K, K), jnp.int32),
        pltpu.VMEM((K, 128), jnp.float32),
    ],
)
def _deg_kernel(dst_hbm, ones_hbm, zeros_hbm, out_hbm, accum, dstbuf, onesbuf):
    c = lax.axis_index("c")
    s = lax.axis_index("s")
    nb = E // (NC * NS) // K
    pltpu.sync_copy(zeros_hbm, accum.at[pl.ds(s * RPT, RPT)])
    pltpu.sync_copy(ones_hbm, onesbuf)
    pltpu.sync_copy(dst_hbm.at[c * NS + s], dstbuf)
    plsc.subcore_barrier()

    def body(j, carry):
        pltpu.sync_copy(onesbuf, accum.at[dstbuf.at[j]], add=True)
        return carry

    lax.fori_loop(0, nb, body, 0)
    plsc.subcore_barrier()
    pltpu.sync_copy(accum.at[pl.ds(s * RPT, RPT)],
                    out_hbm.at[c, pl.ds(s * RPT, RPT)])


def _make_agg(src_rows, split_edges):
    ept = E // (NC * NS) if split_edges else E // NS
    nch = ept // CHUNK

    @functools.partial(
        pl.kernel,
        out_type=jax.ShapeDtypeStruct((NC, NP, 128), jnp.float32),
        mesh=_sc_mesh,
        scratch_types=[
            pltpu.VMEM_SHARED((NP, 128), jnp.float32),
            pltpu.VMEM((CHUNK,), jnp.int32),
            pltpu.VMEM((NBC, K), jnp.int32),
            pltpu.VMEM((K, 128), jnp.float32),
            pltpu.VMEM((K, 128), jnp.float32),
            pltpu.VMEM((K, 128), jnp.float32),
            pltpu.VMEM((K, 128), jnp.float32),
            pltpu.SemaphoreType.DMA,
            pltpu.SemaphoreType.DMA,
        ],
    )
    def agg(feat_hbm, src_hbm, dst_hbm, zeros_hbm, out_hbm,
            accum, srcbuf, dstbuf, rows0, rows1, rows2, rows3, gsem, ssem):
        c = lax.axis_index("c")
        s = lax.axis_index("s")
        pltpu.sync_copy(zeros_hbm, accum.at[pl.ds(s * RPT, RPT)])
        if split_edges:
            tile = c * NS + s
        else:
            tile = s
        plsc.subcore_barrier()

        def wait_gather(buf):
            pltpu.make_async_copy(
                feat_hbm.at[srcbuf.at[pl.ds(0, K)]], buf, gsem).wait()

        def wait_scatter():
            pltpu.make_async_copy(
                rows0, accum.at[dstbuf.at[0]], ssem).wait()

        def chunk_body(q, carry):
            pltpu.sync_copy(src_hbm.at[pl.ds(tile * ept + q * CHUNK, CHUNK)],
                            srcbuf)
            pltpu.sync_copy(dst_hbm.at[tile, q], dstbuf)
            if not split_edges:
                off = c * N

                def addoff(i, carry2):
                    srcbuf[pl.ds(i * 16, 16)] = srcbuf[pl.ds(i * 16, 16)] + off
                    return carry2

                lax.fori_loop(0, CHUNK // 16, addoff, 0)

            bufs = (rows0, rows1, rows2, rows3)
            pltpu.async_copy(feat_hbm.at[srcbuf.at[pl.ds(0, K)]], rows0, gsem)
            pltpu.async_copy(feat_hbm.at[srcbuf.at[pl.ds(K, K)]], rows1, gsem)

            def step(j, carry2):
                def stage(cur, tgt):
                    wait_gather(cur)

                    @pl.when(j + 2 < NBC)
                    def _():
                        @pl.when(j >= 2)
                        def _():
                            wait_scatter()

                        pltpu.async_copy(
                            feat_hbm.at[srcbuf.at[pl.ds((j + 2) * K, K)]],
                            tgt, gsem)

                    pltpu.async_copy(cur, accum.at[dstbuf.at[j]], ssem,
                                     add=True)

                for p in range(4):
                    @pl.when(j % 4 == p)
                    def _(p=p):
                        stage(bufs[p], bufs[(p + 2) % 4])

                return carry2

            lax.fori_loop(0, NBC, step, 0)
            for _ in range(4):
                wait_scatter()
            return carry

        lax.fori_loop(0, nch, chunk_body, 0)
        plsc.subcore_barrier()
        pltpu.sync_copy(accum.at[pl.ds(s * RPT, RPT)],
                        out_hbm.at[c, pl.ds(s * RPT, RPT)])

    return agg


_agg_split = _make_agg(N, True)
_agg_feat = _make_agg(2 * N, False)


_BLK = 1000


def _prescale_call(degp, x):
    def body(degp_ref, x_ref, xs_ref, dis_ref):
        deg = degp_ref[0, :, 0:1] + degp_ref[1, :, 0:1] + 1.0
        d = lax.rsqrt(deg)
        dis_ref[...] = d
        xs_ref[...] = x_ref[...] * d

    return pl.pallas_call(
        body,
        grid=(N // _BLK,),
        in_specs=[
            pl.BlockSpec((NC, _BLK, 128), lambda i: (0, i, 0)),
            pl.BlockSpec((_BLK, 128), lambda i: (i, 0)),
        ],
        out_specs=[
            pl.BlockSpec((_BLK, 128), lambda i: (i, 0)),
            pl.BlockSpec((_BLK, 1), lambda i: (i, 0)),
        ],
        out_shape=[
            jax.ShapeDtypeStruct((N, 128), jnp.float32),
            jax.ShapeDtypeStruct((N, 1), jnp.float32),
        ],
    )(degp, x)


def _layer1_call(s1, x, dis, W1, b1, W2p):
    def body(s1_ref, x_ref, dis_ref, W1_ref, b1_ref, W2_ref, out_ref):
        d = dis_ref[...]
        agg = d * (s1_ref[0] + s1_ref[1]) + (d * d) * x_ref[...]
        h1 = jnp.maximum(
            jnp.dot(agg, W1_ref[...], preferred_element_type=jnp.float32)
            + b1_ref[...], 0.0)
        g = jnp.dot(h1, W2_ref[...], preferred_element_type=jnp.float32)
        out_ref[...] = d * g

    return pl.pallas_call(
        body,
        grid=(N // _BLK, 2),
        in_specs=[
            pl.BlockSpec((NC, _BLK, 128), lambda i, j: (0, i, 0)),
            pl.BlockSpec((_BLK, 128), lambda i, j: (i, 0)),
            pl.BlockSpec((_BLK, 1), lambda i, j: (i, 0)),
            pl.BlockSpec((128, 512), lambda i, j: (0, 0)),
            pl.BlockSpec((1, 512), lambda i, j: (0, 0)),
            pl.BlockSpec((512, 128), lambda i, j: (0, j)),
        ],
        out_specs=pl.BlockSpec((_BLK, 128),
                               lambda i, j: (j * (N // _BLK) + i, 0)),
        out_shape=jax.ShapeDtypeStruct((2 * N, 128), jnp.float32),
    )(s1, x, dis, W1, b1, W2p)


def _layer2_call(s2, gsc, dis, b2p):
    def body(s2_ref, gsc_ref, dis_ref, b2_ref, out_ref):
        d = dis_ref[...]
        b = jnp.where(pl.program_id(1) == 0, b2_ref[0:1, :], b2_ref[1:2, :])
        out_ref[...] = jnp.maximum(
            d * (s2_ref[0] + gsc_ref[...]) + b, 0.0)

    return pl.pallas_call(
        body,
        grid=(N // _BLK, 2),
        in_specs=[
            pl.BlockSpec((1, _BLK, 128), lambda i, j: (j, i, 0)),
            pl.BlockSpec((_BLK, 128), lambda i, j: (j * (N // _BLK) + i, 0)),
            pl.BlockSpec((_BLK, 1), lambda i, j: (i, 0)),
            pl.BlockSpec((2, 128), lambda i, j: (0, 0)),
        ],
        out_specs=pl.BlockSpec((_BLK, 128), lambda i, j: (i, j)),
        out_shape=jax.ShapeDtypeStruct((N, 256), jnp.float32),
    )(s2, gsc, dis, b2p)


def kernel(x, edge_index, W1, b1, W2, b2):
    src = edge_index[0]
    dst3s = edge_index[1].reshape(NC * NS, E // (NC * NS) // CHUNK, NBC, K)
    dst3f = edge_index[1].reshape(NS, E // NS // CHUNK, NBC, K)
    dst3d = edge_index[1].reshape(NC * NS, E // (NC * NS) // K, K)
    ones128 = jnp.ones((K, 128), jnp.float32)
    zeros128 = jnp.zeros((RPT, 128), jnp.float32)
    W2p = jnp.pad(W2, ((0, 0), (0, 256 - W2.shape[1])))
    b2p = jnp.pad(b2, (0, 256 - b2.shape[0])).reshape(2, 128)

    degp = _deg_kernel(dst3d, ones128, zeros128)
    xs, dis = _prescale_call(degp, x)
    s1 = _agg_split(xs, src, dst3s, zeros128)
    gsc = _layer1_call(s1, x, dis, W1, b1.reshape(1, 512), W2p)
    s2 = _agg_feat(gsc, src, dst3f, zeros128)
    o = _layer2_call(s2, gsc, dis, b2p)
    return o[:, :250]


# speedup vs baseline: 29.4431x; 1.0029x over previous
import functools

import jax
import jax.numpy as jnp
from jax import lax
from jax.experimental import pallas as pl
from jax.experimental.pallas import tpu as pltpu
from jax.experimental.pallas import tpu_sc as plsc

N = 10000
E = 320000
K = 80
NC, NS = 2, 16
NP = 10240
RPT = NP // NS
CHUNK = 2000
NBC = CHUNK // K

_sc_mesh = plsc.VectorSubcoreMesh(core_axis_name="c", subcore_axis_name="s")


@functools.partial(
    pl.kernel,
    out_type=jax.ShapeDtypeStruct((NC, NP, 128), jnp.float32),
    mesh=_sc_mesh,
    scratch_types=[
        pltpu.VMEM_SHARED((NP, 128), jnp.float32),
        pltpu.VMEM((E // (NC * NS) // K, K), jnp.int32),
        pltpu.VMEM((K, 128), jnp.float32),
    ],
)
def _deg_kernel(dst_hbm, ones_hbm, zeros_hbm, out_hbm, accum, dstbuf, onesbuf):
    c = lax.axis_index("c")
    s = lax.axis_index("s")
    nb = E // (NC * NS) // K
    pltpu.sync_copy(zeros_hbm, accum.at[pl.ds(s * RPT, RPT)])
    pltpu.sync_copy(ones_hbm, onesbuf)
    pltpu.sync_copy(dst_hbm.at[c * NS + s], dstbuf)
    plsc.subcore_barrier()

    def body(j, carry):
        pltpu.sync_copy(onesbuf, accum.at[dstbuf.at[j]], add=True)
        return carry

    lax.fori_loop(0, nb, body, 0)
    plsc.subcore_barrier()
    pltpu.sync_copy(accum.at[pl.ds(s * RPT, RPT)],
                    out_hbm.at[c, pl.ds(s * RPT, RPT)])


def _make_agg(src_rows, split_edges):
    ept = E // (NC * NS) if split_edges else E // NS
    nch = ept // CHUNK

    @functools.partial(
        pl.kernel,
        out_type=jax.ShapeDtypeStruct((NC, NP, 128), jnp.float32),
        mesh=_sc_mesh,
        scratch_types=[
            pltpu.VMEM_SHARED((NP, 128), jnp.float32),
            pltpu.VMEM((CHUNK,), jnp.int32),
            pltpu.VMEM((NBC, K), jnp.int32),
            pltpu.VMEM((K, 128), jnp.float32),
            pltpu.VMEM((K, 128), jnp.float32),
            pltpu.VMEM((K, 128), jnp.float32),
            pltpu.VMEM((K, 128), jnp.float32),
            pltpu.SemaphoreType.DMA,
            pltpu.SemaphoreType.DMA,
        ],
    )
    def agg(feat_hbm, src_hbm, dst_hbm, zeros_hbm, out_hbm,
            accum, srcbuf, dstbuf, rows0, rows1, rows2, rows3, gsem, ssem):
        c = lax.axis_index("c")
        s = lax.axis_index("s")
        pltpu.sync_copy(zeros_hbm, accum.at[pl.ds(s * RPT, RPT)])
        if split_edges:
            tile = c * NS + s
        else:
            tile = s
        plsc.subcore_barrier()

        def wait_gather(buf):
            pltpu.make_async_copy(
                feat_hbm.at[srcbuf.at[pl.ds(0, K)]], buf, gsem).wait()

        def wait_scatter():
            pltpu.make_async_copy(
                rows0, accum.at[dstbuf.at[0]], ssem).wait()

        def chunk_body(q, carry):
            pltpu.sync_copy(src_hbm.at[pl.ds(tile * ept + q * CHUNK, CHUNK)],
                            srcbuf)
            pltpu.sync_copy(dst_hbm.at[tile, q], dstbuf)
            if not split_edges:
                off = c * N

                def addoff(i, carry2):
                    srcbuf[pl.ds(i * 16, 16)] = srcbuf[pl.ds(i * 16, 16)] + off
                    return carry2

                lax.fori_loop(0, CHUNK // 16, addoff, 0)

            bufs = (rows0, rows1, rows2, rows3)
            pltpu.async_copy(feat_hbm.at[srcbuf.at[pl.ds(0, K)]], rows0, gsem)
            pltpu.async_copy(feat_hbm.at[srcbuf.at[pl.ds(K, K)]], rows1, gsem)

            def step(j, carry2):
                def stage(cur, tgt):
                    wait_gather(cur)

                    @pl.when(j + 2 < NBC)
                    def _():
                        @pl.when(j >= 2)
                        def _():
                            wait_scatter()

                        pltpu.async_copy(
                            feat_hbm.at[srcbuf.at[pl.ds((j + 2) * K, K)]],
                            tgt, gsem)

                    pltpu.async_copy(cur, accum.at[dstbuf.at[j]], ssem,
                                     add=True)

                for p in range(4):
                    @pl.when(j % 4 == p)
                    def _(p=p):
                        stage(bufs[p], bufs[(p + 2) % 4])

                return carry2

            lax.fori_loop(0, NBC, step, 0)
            for _ in range(4):
                wait_scatter()
            return carry

        lax.fori_loop(0, nch, chunk_body, 0)
        plsc.subcore_barrier()
        pltpu.sync_copy(accum.at[pl.ds(s * RPT, RPT)],
                        out_hbm.at[c, pl.ds(s * RPT, RPT)])

    return agg


_agg_split = _make_agg(N, True)
_agg_feat = _make_agg(2 * N, False)


_BLK = 1000


def _prescale_call(degp, x):
    def body(degp_ref, x_ref, xs_ref, dis_ref):
        deg = degp_ref[0, :, 0:1] + degp_ref[1, :, 0:1] + 1.0
        d = lax.rsqrt(deg)
        dis_ref[...] = d
        xs_ref[...] = x_ref[...] * d

    return pl.pallas_call(
        body,
        grid=(N // _BLK,),
        in_specs=[
            pl.BlockSpec((NC, _BLK, 128), lambda i: (0, i, 0)),
            pl.BlockSpec((_BLK, 128), lambda i: (i, 0)),
        ],
        out_specs=[
            pl.BlockSpec((_BLK, 128), lambda i: (i, 0)),
            pl.BlockSpec((_BLK, 1), lambda i: (i, 0)),
        ],
        out_shape=[
            jax.ShapeDtypeStruct((N, 128), jnp.float32),
            jax.ShapeDtypeStruct((N, 1), jnp.float32),
        ],
    )(degp, x)


def _layer1_call(s1, x, dis, W1, b1, W2p):
    def body(s1_ref, x_ref, dis_ref, W1_ref, b1_ref, W2_ref, out_ref):
        d = dis_ref[...]
        agg = d * (s1_ref[0] + s1_ref[1]) + (d * d) * x_ref[...]
        h1 = jnp.maximum(
            jnp.dot(agg.astype(jnp.bfloat16), W1_ref[...],
                    preferred_element_type=jnp.float32)
            + b1_ref[...], 0.0)
        g = jnp.dot(h1.astype(jnp.bfloat16), W2_ref[...],
                    preferred_element_type=jnp.float32)
        out_ref[...] = d * g

    return pl.pallas_call(
        body,
        grid=(N // _BLK, 2),
        in_specs=[
            pl.BlockSpec((NC, _BLK, 128), lambda i, j: (0, i, 0)),
            pl.BlockSpec((_BLK, 128), lambda i, j: (i, 0)),
            pl.BlockSpec((_BLK, 1), lambda i, j: (i, 0)),
            pl.BlockSpec((128, 512), lambda i, j: (0, 0)),
            pl.BlockSpec((1, 512), lambda i, j: (0, 0)),
            pl.BlockSpec((512, 128), lambda i, j: (0, j)),
        ],
        out_specs=pl.BlockSpec((_BLK, 128),
                               lambda i, j: (j * (N // _BLK) + i, 0)),
        out_shape=jax.ShapeDtypeStruct((2 * N, 128), jnp.float32),
    )(s1, x, dis, W1.astype(jnp.bfloat16), b1, W2p.astype(jnp.bfloat16))


def _layer2_call(s2, gsc, dis, b2p):
    def body(s2_ref, gsc_ref, dis_ref, b2_ref, out_ref):
        d = dis_ref[...]
        b = jnp.where(pl.program_id(1) == 0, b2_ref[0:1, :], b2_ref[1:2, :])
        out_ref[...] = jnp.maximum(
            d * (s2_ref[0] + gsc_ref[...]) + b, 0.0)

    return pl.pallas_call(
        body,
        grid=(N // _BLK, 2),
        in_specs=[
            pl.BlockSpec((1, _BLK, 128), lambda i, j: (j, i, 0)),
            pl.BlockSpec((_BLK, 128), lambda i, j: (j * (N // _BLK) + i, 0)),
            pl.BlockSpec((_BLK, 1), lambda i, j: (i, 0)),
            pl.BlockSpec((2, 128), lambda i, j: (0, 0)),
        ],
        out_specs=pl.BlockSpec((_BLK, 128), lambda i, j: (i, j)),
        out_shape=jax.ShapeDtypeStruct((N, 256), jnp.float32),
    )(s2, gsc, dis, b2p)


def kernel(x, edge_index, W1, b1, W2, b2):
    src = edge_index[0]
    dst3s = edge_index[1].reshape(NC * NS, E // (NC * NS) // CHUNK, NBC, K)
    dst3f = edge_index[1].reshape(NS, E // NS // CHUNK, NBC, K)
    dst3d = edge_index[1].reshape(NC * NS, E // (NC * NS) // K, K)
    ones128 = jnp.ones((K, 128), jnp.float32)
    zeros128 = jnp.zeros((RPT, 128), jnp.float32)
    W2p = jnp.pad(W2, ((0, 0), (0, 256 - W2.shape[1])))
    b2p = jnp.pad(b2, (0, 256 - b2.shape[0])).reshape(2, 128)

    degp = _deg_kernel(dst3d, ones128, zeros128)
    xs, dis = _prescale_call(degp, x)
    s1 = _agg_split(xs, src, dst3s, zeros128)
    gsc = _layer1_call(s1, x, dis, W1, b1.reshape(1, 512), W2p)
    s2 = _agg_feat(gsc, src, dst3f, zeros128)
    o = _layer2_call(s2, gsc, dis, b2p)
    return o[:, :250]


# speedup vs baseline: 29.9974x; 1.0188x over previous
import functools

import jax
import jax.numpy as jnp
from jax import lax
from jax.experimental import pallas as pl
from jax.experimental.pallas import tpu as pltpu
from jax.experimental.pallas import tpu_sc as plsc

N = 10000
E = 320000
K = 80
NC, NS = 2, 16
NP = 10240
RPT = NP // NS
CHUNK = 2000
NBC = CHUNK // K

_sc_mesh = plsc.VectorSubcoreMesh(core_axis_name="c", subcore_axis_name="s")


@functools.partial(
    pl.kernel,
    out_type=jax.ShapeDtypeStruct((NC, NP, 128), jnp.float32),
    mesh=_sc_mesh,
    scratch_types=[
        pltpu.VMEM_SHARED((NP, 128), jnp.float32),
        pltpu.VMEM((E // (NC * NS) // K, K), jnp.int32),
        pltpu.VMEM((K, 128), jnp.float32),
        pltpu.SemaphoreType.DMA,
    ],
)
def _deg_kernel(dst_hbm, ones_hbm, zeros_hbm, out_hbm,
                accum, dstbuf, onesbuf, sem):
    c = lax.axis_index("c")
    s = lax.axis_index("s")
    nb = E // (NC * NS) // K
    pltpu.async_copy(zeros_hbm, accum.at[pl.ds(s * RPT, RPT)], sem)
    pltpu.async_copy(ones_hbm, onesbuf, sem)
    pltpu.async_copy(dst_hbm.at[c * NS + s], dstbuf, sem)
    pltpu.make_async_copy(zeros_hbm, accum.at[pl.ds(0, RPT)], sem).wait()
    pltpu.make_async_copy(ones_hbm, onesbuf, sem).wait()
    pltpu.make_async_copy(dst_hbm.at[0], dstbuf, sem).wait()
    plsc.subcore_barrier()

    def body(j, carry):
        pltpu.sync_copy(onesbuf, accum.at[dstbuf.at[j]], add=True)
        return carry

    lax.fori_loop(0, nb, body, 0)
    plsc.subcore_barrier()
    pltpu.sync_copy(accum.at[pl.ds(s * RPT, RPT)],
                    out_hbm.at[c, pl.ds(s * RPT, RPT)])


def _make_agg(src_rows, split_edges):
    ept = E // (NC * NS) if split_edges else E // NS
    nch = ept // CHUNK

    @functools.partial(
        pl.kernel,
        out_type=jax.ShapeDtypeStruct((NC, NP, 128), jnp.float32),
        mesh=_sc_mesh,
        scratch_types=[
            pltpu.VMEM_SHARED((NP, 128), jnp.float32),
            pltpu.VMEM((CHUNK,), jnp.int32),
            pltpu.VMEM((NBC, K), jnp.int32),
            pltpu.VMEM((K, 128), jnp.float32),
            pltpu.VMEM((K, 128), jnp.float32),
            pltpu.VMEM((K, 128), jnp.float32),
            pltpu.VMEM((K, 128), jnp.float32),
            pltpu.SemaphoreType.DMA,
            pltpu.SemaphoreType.DMA,
        ],
    )
    def agg(feat_hbm, src_hbm, dst_hbm, zeros_hbm, out_hbm,
            accum, srcbuf, dstbuf, rows0, rows1, rows2, rows3, gsem, ssem):
        c = lax.axis_index("c")
        s = lax.axis_index("s")
        pltpu.async_copy(zeros_hbm, accum.at[pl.ds(s * RPT, RPT)], ssem)
        if split_edges:
            tile = c * NS + s
        else:
            tile = s

        def wait_gather(buf):
            pltpu.make_async_copy(
                feat_hbm.at[srcbuf.at[pl.ds(0, K)]], buf, gsem).wait()

        def wait_scatter():
            pltpu.make_async_copy(
                rows0, accum.at[dstbuf.at[0]], ssem).wait()

        def chunk_body(q, carry):
            pltpu.async_copy(src_hbm.at[pl.ds(tile * ept + q * CHUNK, CHUNK)],
                             srcbuf, gsem)
            pltpu.async_copy(dst_hbm.at[tile, q], dstbuf, gsem)
            pltpu.make_async_copy(
                src_hbm.at[pl.ds(0, CHUNK)], srcbuf, gsem).wait()
            pltpu.make_async_copy(dst_hbm.at[0, 0], dstbuf, gsem).wait()

            @pl.when(q == 0)
            def _():
                pltpu.make_async_copy(
                    zeros_hbm, accum.at[pl.ds(0, RPT)], ssem).wait()
                plsc.subcore_barrier()

            if not split_edges:
                off = c * N

                def addoff(i, carry2):
                    srcbuf[pl.ds(i * 16, 16)] = srcbuf[pl.ds(i * 16, 16)] + off
                    return carry2

                lax.fori_loop(0, CHUNK // 16, addoff, 0)

            bufs = (rows0, rows1, rows2, rows3)
            pltpu.async_copy(feat_hbm.at[srcbuf.at[pl.ds(0, K)]], rows0, gsem)
            pltpu.async_copy(feat_hbm.at[srcbuf.at[pl.ds(K, K)]], rows1, gsem)

            def step(j, carry2):
                def stage(cur, tgt):
                    wait_gather(cur)

                    @pl.when(j + 2 < NBC)
                    def _():
                        @pl.when(j >= 2)
                        def _():
                            wait_scatter()

                        pltpu.async_copy(
                            feat_hbm.at[srcbuf.at[pl.ds((j + 2) * K, K)]],
                            tgt, gsem)

                    pltpu.async_copy(cur, accum.at[dstbuf.at[j]], ssem,
                                     add=True)

                for p in range(4):
                    @pl.when(j % 4 == p)
                    def _(p=p):
                        stage(bufs[p], bufs[(p + 2) % 4])

                return carry2

            lax.fori_loop(0, NBC, step, 0)
            for _ in range(4):
                wait_scatter()
            return carry

        lax.fori_loop(0, nch, chunk_body, 0)
        plsc.subcore_barrier()
        pltpu.sync_copy(accum.at[pl.ds(s * RPT, RPT)],
                        out_hbm.at[c, pl.ds(s * RPT, RPT)])

    return agg


_agg_split = _make_agg(N, True)
_agg_feat = _make_agg(2 * N, False)


_BLK = 1000


def _prescale_call(degp, x):
    def body(degp_ref, x_ref, xs_ref, dis_ref):
        deg = degp_ref[0, :, 0:1] + degp_ref[1, :, 0:1] + 1.0
        d = lax.rsqrt(deg)
        dis_ref[...] = d
        xs_ref[...] = x_ref[...] * d

    return pl.pallas_call(
        body,
        grid=(N // _BLK,),
        in_specs=[
            pl.BlockSpec((NC, _BLK, 128), lambda i: (0, i, 0)),
            pl.BlockSpec((_BLK, 128), lambda i: (i, 0)),
        ],
        out_specs=[
            pl.BlockSpec((_BLK, 128), lambda i: (i, 0)),
            pl.BlockSpec((_BLK, 1), lambda i: (i, 0)),
        ],
        out_shape=[
            jax.ShapeDtypeStruct((N, 128), jnp.float32),
            jax.ShapeDtypeStruct((N, 1), jnp.float32),
        ],
    )(degp, x)


def _layer1_call(s1, x, dis, W1, b1, W2p):
    def body(s1_ref, x_ref, dis_ref, W1_ref, b1_ref, W2_ref, out_ref):
        d = dis_ref[...]
        agg = d * (s1_ref[0] + s1_ref[1]) + (d * d) * x_ref[...]
        h1 = jnp.maximum(
            jnp.dot(agg.astype(jnp.bfloat16), W1_ref[...],
                    preferred_element_type=jnp.float32)
            + b1_ref[...], 0.0)
        g = jnp.dot(h1.astype(jnp.bfloat16), W2_ref[...],
                    preferred_element_type=jnp.float32)
        out_ref[...] = d * g

    return pl.pallas_call(
        body,
        grid=(N // _BLK, 2),
        in_specs=[
            pl.BlockSpec((NC, _BLK, 128), lambda i, j: (0, i, 0)),
            pl.BlockSpec((_BLK, 128), lambda i, j: (i, 0)),
            pl.BlockSpec((_BLK, 1), lambda i, j: (i, 0)),
            pl.BlockSpec((128, 512), lambda i, j: (0, 0)),
            pl.BlockSpec((1, 512), lambda i, j: (0, 0)),
            pl.BlockSpec((512, 128), lambda i, j: (0, j)),
        ],
        out_specs=pl.BlockSpec((_BLK, 128),
                               lambda i, j: (j * (N // _BLK) + i, 0)),
        out_shape=jax.ShapeDtypeStruct((2 * N, 128), jnp.float32),
    )(s1, x, dis, W1.astype(jnp.bfloat16), b1, W2p.astype(jnp.bfloat16))


def _layer2_call(s2, gsc, dis, b2p):
    def body(s2_ref, gsc_ref, dis_ref, b2_ref, out_ref):
        d = dis_ref[...]
        b = jnp.where(pl.program_id(1) == 0, b2_ref[0:1, :], b2_ref[1:2, :])
        out_ref[...] = jnp.maximum(
            d * (s2_ref[0] + gsc_ref[...]) + b, 0.0)

    return pl.pallas_call(
        body,
        grid=(N // _BLK, 2),
        in_specs=[
            pl.BlockSpec((1, _BLK, 128), lambda i, j: (j, i, 0)),
            pl.BlockSpec((_BLK, 128), lambda i, j: (j * (N // _BLK) + i, 0)),
            pl.BlockSpec((_BLK, 1), lambda i, j: (i, 0)),
            pl.BlockSpec((2, 128), lambda i, j: (0, 0)),
        ],
        out_specs=pl.BlockSpec((_BLK, 128), lambda i, j: (i, j)),
        out_shape=jax.ShapeDtypeStruct((N, 256), jnp.float32),
    )(s2, gsc, dis, b2p)


def kernel(x, edge_index, W1, b1, W2, b2):
    src = edge_index[0]
    dst3s = edge_index[1].reshape(NC * NS, E // (NC * NS) // CHUNK, NBC, K)
    dst3f = edge_index[1].reshape(NS, E // NS // CHUNK, NBC, K)
    dst3d = edge_index[1].reshape(NC * NS, E // (NC * NS) // K, K)
    ones128 = jnp.ones((K, 128), jnp.float32)
    zeros128 = jnp.zeros((RPT, 128), jnp.float32)
    W2p = jnp.pad(W2, ((0, 0), (0, 256 - W2.shape[1])))
    b2p = jnp.pad(b2, (0, 256 - b2.shape[0])).reshape(2, 128)

    degp = _deg_kernel(dst3d, ones128, zeros128)
    xs, dis = _prescale_call(degp, x)
    s1 = _agg_split(xs, src, dst3s, zeros128)
    gsc = _layer1_call(s1, x, dis, W1, b1.reshape(1, 512), W2p)
    s2 = _agg_feat(gsc, src, dst3f, zeros128)
    o = _layer2_call(s2, gsc, dis, b2p)
    return o[:, :250]


# speedup vs baseline: 30.0885x; 1.0030x over previous
import functools

import jax
import jax.numpy as jnp
from jax import lax
from jax.experimental import pallas as pl
from jax.experimental.pallas import tpu as pltpu
from jax.experimental.pallas import tpu_sc as plsc

N = 10000
E = 320000
K = 80
NC, NS = 2, 16
NP = 10240
RPT = NP // NS
CHUNK = 2000
NBC = CHUNK // K

_sc_mesh = plsc.VectorSubcoreMesh(core_axis_name="c", subcore_axis_name="s")


@functools.partial(
    pl.kernel,
    out_type=jax.ShapeDtypeStruct((NC, NP, 128), jnp.float32),
    mesh=_sc_mesh,
    scratch_types=[
        pltpu.VMEM_SHARED((NP, 128), jnp.float32),
        pltpu.VMEM((E // (NC * NS) // K, K), jnp.int32),
        pltpu.VMEM((K, 128), jnp.float32),
        pltpu.SemaphoreType.DMA,
    ],
)
def _deg_kernel(dst_hbm, ones_hbm, zeros_hbm, out_hbm,
                accum, dstbuf, onesbuf, sem):
    c = lax.axis_index("c")
    s = lax.axis_index("s")
    nb = E // (NC * NS) // K
    pltpu.async_copy(zeros_hbm, accum.at[pl.ds(s * RPT, RPT)], sem)
    pltpu.async_copy(ones_hbm, onesbuf, sem)
    pltpu.async_copy(dst_hbm.at[c * NS + s], dstbuf, sem)
    pltpu.make_async_copy(zeros_hbm, accum.at[pl.ds(0, RPT)], sem).wait()
    pltpu.make_async_copy(ones_hbm, onesbuf, sem).wait()
    pltpu.make_async_copy(dst_hbm.at[0], dstbuf, sem).wait()
    plsc.subcore_barrier()

    def body(j, carry):
        pltpu.async_copy(onesbuf, accum.at[dstbuf.at[j]], sem, add=True)

        @pl.when(j >= 4)
        def _():
            pltpu.make_async_copy(onesbuf, accum.at[dstbuf.at[0]], sem).wait()

        return carry

    lax.fori_loop(0, nb, body, 0)
    for _ in range(4):
        pltpu.make_async_copy(onesbuf, accum.at[dstbuf.at[0]], sem).wait()
    plsc.subcore_barrier()
    pltpu.sync_copy(accum.at[pl.ds(s * RPT, RPT)],
                    out_hbm.at[c, pl.ds(s * RPT, RPT)])


def _make_agg(src_rows, split_edges):
    ept = E // (NC * NS) if split_edges else E // NS
    nch = ept // CHUNK

    @functools.partial(
        pl.kernel,
        out_type=jax.ShapeDtypeStruct((NC, NP, 128), jnp.float32),
        mesh=_sc_mesh,
        scratch_types=[
            pltpu.VMEM_SHARED((NP, 128), jnp.float32),
            pltpu.VMEM((CHUNK,), jnp.int32),
            pltpu.VMEM((NBC, K), jnp.int32),
            pltpu.VMEM((K, 128), jnp.float32),
            pltpu.VMEM((K, 128), jnp.float32),
            pltpu.VMEM((K, 128), jnp.float32),
            pltpu.VMEM((K, 128), jnp.float32),
            pltpu.SemaphoreType.DMA,
            pltpu.SemaphoreType.DMA,
        ],
    )
    def agg(feat_hbm, src_hbm, dst_hbm, zeros_hbm, out_hbm,
            accum, srcbuf, dstbuf, rows0, rows1, rows2, rows3, gsem, ssem):
        c = lax.axis_index("c")
        s = lax.axis_index("s")
        pltpu.async_copy(zeros_hbm, accum.at[pl.ds(s * RPT, RPT)], ssem)
        if split_edges:
            tile = c * NS + s
        else:
            tile = s

        def wait_gather(buf):
            pltpu.make_async_copy(
                feat_hbm.at[srcbuf.at[pl.ds(0, K)]], buf, gsem).wait()

        def wait_scatter():
            pltpu.make_async_copy(
                rows0, accum.at[dstbuf.at[0]], ssem).wait()

        def chunk_body(q, carry):
            pltpu.async_copy(src_hbm.at[pl.ds(tile * ept + q * CHUNK, CHUNK)],
                             srcbuf, gsem)
            pltpu.async_copy(dst_hbm.at[tile, q], dstbuf, gsem)
            pltpu.make_async_copy(
                src_hbm.at[pl.ds(0, CHUNK)], srcbuf, gsem).wait()
            pltpu.make_async_copy(dst_hbm.at[0, 0], dstbuf, gsem).wait()

            @pl.when(q == 0)
            def _():
                pltpu.make_async_copy(
                    zeros_hbm, accum.at[pl.ds(0, RPT)], ssem).wait()
                plsc.subcore_barrier()

            if not split_edges:
                off = c * N

                def addoff(i, carry2):
                    srcbuf[pl.ds(i * 16, 16)] = srcbuf[pl.ds(i * 16, 16)] + off
                    return carry2

                lax.fori_loop(0, CHUNK // 16, addoff, 0)

            bufs = (rows0, rows1, rows2, rows3)
            pltpu.async_copy(feat_hbm.at[srcbuf.at[pl.ds(0, K)]], rows0, gsem)
            pltpu.async_copy(feat_hbm.at[srcbuf.at[pl.ds(K, K)]], rows1, gsem)

            def step(j, carry2):
                def stage(cur, tgt):
                    wait_gather(cur)

                    @pl.when(j + 2 < NBC)
                    def _():
                        @pl.when(j >= 2)
                        def _():
                            wait_scatter()

                        pltpu.async_copy(
                            feat_hbm.at[srcbuf.at[pl.ds((j + 2) * K, K)]],
                            tgt, gsem)

                    pltpu.async_copy(cur, accum.at[dstbuf.at[j]], ssem,
                                     add=True)

                for p in range(4):
                    @pl.when(j % 4 == p)
                    def _(p=p):
                        stage(bufs[p], bufs[(p + 2) % 4])

                return carry2

            lax.fori_loop(0, NBC, step, 0)
            for _ in range(4):
                wait_scatter()
            return carry

        lax.fori_loop(0, nch, chunk_body, 0)
        plsc.subcore_barrier()
        pltpu.sync_copy(accum.at[pl.ds(s * RPT, RPT)],
                        out_hbm.at[c, pl.ds(s * RPT, RPT)])

    return agg


_agg_split = _make_agg(N, True)
_agg_feat = _make_agg(2 * N, False)


_BLK = 1000


def _prescale_call(degp, x):
    def body(degp_ref, x_ref, xs_ref, dis_ref):
        deg = degp_ref[0, :, 0:1] + degp_ref[1, :, 0:1] + 1.0
        d = lax.rsqrt(deg)
        dis_ref[...] = d
        xs_ref[...] = x_ref[...] * d

    return pl.pallas_call(
        body,
        grid=(N // _BLK,),
        in_specs=[
            pl.BlockSpec((NC, _BLK, 128), lambda i: (0, i, 0)),
            pl.BlockSpec((_BLK, 128), lambda i: (i, 0)),
        ],
        out_specs=[
            pl.BlockSpec((_BLK, 128), lambda i: (i, 0)),
            pl.BlockSpec((_BLK, 1), lambda i: (i, 0)),
        ],
        out_shape=[
            jax.ShapeDtypeStruct((N, 128), jnp.float32),
            jax.ShapeDtypeStruct((N, 1), jnp.float32),
        ],
    )(degp, x)


def _layer1_call(s1, x, dis, W1, b1, W2p):
    def body(s1_ref, x_ref, dis_ref, W1_ref, b1_ref, W2_ref, out_ref):
        d = dis_ref[...]
        agg = d * (s1_ref[0] + s1_ref[1]) + (d * d) * x_ref[...]
        h1 = jnp.maximum(
            jnp.dot(agg.astype(jnp.bfloat16), W1_ref[...],
                    preferred_element_type=jnp.float32)
            + b1_ref[...], 0.0)
        g = jnp.dot(h1.astype(jnp.bfloat16), W2_ref[...],
                    preferred_element_type=jnp.float32)
        out_ref[...] = d * g

    return pl.pallas_call(
        body,
        grid=(N // _BLK, 2),
        in_specs=[
            pl.BlockSpec((NC, _BLK, 128), lambda i, j: (0, i, 0)),
            pl.BlockSpec((_BLK, 128), lambda i, j: (i, 0)),
            pl.BlockSpec((_BLK, 1), lambda i, j: (i, 0)),
            pl.BlockSpec((128, 512), lambda i, j: (0, 0)),
            pl.BlockSpec((1, 512), lambda i, j: (0, 0)),
            pl.BlockSpec((512, 128), lambda i, j: (0, j)),
        ],
        out_specs=pl.BlockSpec((_BLK, 128),
                               lambda i, j: (j * (N // _BLK) + i, 0)),
        out_shape=jax.ShapeDtypeStruct((2 * N, 128), jnp.float32),
    )(s1, x, dis, W1.astype(jnp.bfloat16), b1, W2p.astype(jnp.bfloat16))


def _layer2_call(s2, gsc, dis, b2p):
    def body(s2_ref, gsc_ref, dis_ref, b2_ref, out_ref):
        d = dis_ref[...]
        b = jnp.where(pl.program_id(1) == 0, b2_ref[0:1, :], b2_ref[1:2, :])
        out_ref[...] = jnp.maximum(
            d * (s2_ref[0] + gsc_ref[...]) + b, 0.0)

    return pl.pallas_call(
        body,
        grid=(N // _BLK, 2),
        in_specs=[
            pl.BlockSpec((1, _BLK, 128), lambda i, j: (j, i, 0)),
            pl.BlockSpec((_BLK, 128), lambda i, j: (j * (N // _BLK) + i, 0)),
            pl.BlockSpec((_BLK, 1), lambda i, j: (i, 0)),
            pl.BlockSpec((2, 128), lambda i, j: (0, 0)),
        ],
        out_specs=pl.BlockSpec((_BLK, 128), lambda i, j: (i, j)),
        out_shape=jax.ShapeDtypeStruct((N, 256), jnp.float32),
    )(s2, gsc, dis, b2p)


def kernel(x, edge_index, W1, b1, W2, b2):
    src = edge_index[0]
    dst3s = edge_index[1].reshape(NC * NS, E // (NC * NS) // CHUNK, NBC, K)
    dst3f = edge_index[1].reshape(NS, E // NS // CHUNK, NBC, K)
    dst3d = edge_index[1].reshape(NC * NS, E // (NC * NS) // K, K)
    ones128 = jnp.ones((K, 128), jnp.float32)
    zeros128 = jnp.zeros((RPT, 128), jnp.float32)
    W2p = jnp.pad(W2, ((0, 0), (0, 256 - W2.shape[1])))
    b2p = jnp.pad(b2, (0, 256 - b2.shape[0])).reshape(2, 128)

    degp = _deg_kernel(dst3d, ones128, zeros128)
    xs, dis = _prescale_call(degp, x)
    s1 = _agg_split(xs, src, dst3s, zeros128)
    gsc = _layer1_call(s1, x, dis, W1, b1.reshape(1, 512), W2p)
    s2 = _agg_feat(gsc, src, dst3f, zeros128)
    o = _layer2_call(s2, gsc, dis, b2p)
    return o[:, :250]


# speedup vs baseline: 30.3006x; 1.0071x over previous
import functools

import jax
import jax.numpy as jnp
from jax import lax
from jax.experimental import pallas as pl
from jax.experimental.pallas import tpu as pltpu
from jax.experimental.pallas import tpu_sc as plsc

N = 10000
E = 320000
K = 80
NC, NS = 2, 16
NP = 10240
RPT = NP // NS
CHUNK = 2000
NBC = CHUNK // K

_sc_mesh = plsc.VectorSubcoreMesh(core_axis_name="c", subcore_axis_name="s")


@functools.partial(
    pl.kernel,
    out_type=jax.ShapeDtypeStruct((NC, NP, 128), jnp.float32),
    mesh=_sc_mesh,
    scratch_types=[
        pltpu.VMEM_SHARED((NP, 128), jnp.float32),
        pltpu.VMEM((E // (NC * NS) // K, K), jnp.int32),
        pltpu.VMEM((K, 128), jnp.float32),
        pltpu.SemaphoreType.DMA,
    ],
)
def _deg_kernel(dst_hbm, ones_hbm, zeros_hbm, out_hbm,
                accum, dstbuf, onesbuf, sem):
    c = lax.axis_index("c")
    s = lax.axis_index("s")
    nb = E // (NC * NS) // K
    pltpu.async_copy(zeros_hbm, accum.at[pl.ds(s * RPT, RPT)], sem)
    pltpu.async_copy(ones_hbm, onesbuf, sem)
    pltpu.async_copy(dst_hbm.at[c * NS + s], dstbuf, sem)
    pltpu.make_async_copy(zeros_hbm, accum.at[pl.ds(0, RPT)], sem).wait()
    pltpu.make_async_copy(ones_hbm, onesbuf, sem).wait()
    pltpu.make_async_copy(dst_hbm.at[0], dstbuf, sem).wait()
    plsc.subcore_barrier()

    def body(j, carry):
        pltpu.async_copy(onesbuf, accum.at[dstbuf.at[j]], sem, add=True)

        @pl.when(j >= 4)
        def _():
            pltpu.make_async_copy(onesbuf, accum.at[dstbuf.at[0]], sem).wait()

        return carry

    lax.fori_loop(0, nb, body, 0)
    for _ in range(4):
        pltpu.make_async_copy(onesbuf, accum.at[dstbuf.at[0]], sem).wait()
    plsc.subcore_barrier()
    pltpu.sync_copy(accum.at[pl.ds(s * RPT, RPT)],
                    out_hbm.at[c, pl.ds(s * RPT, RPT)])


def _make_agg(src_rows, split_edges):
    ept = E // (NC * NS) if split_edges else E // NS
    nch = ept // CHUNK

    @functools.partial(
        pl.kernel,
        out_type=jax.ShapeDtypeStruct((NC, NP, 128), jnp.float32),
        mesh=_sc_mesh,
        scratch_types=[
            pltpu.VMEM_SHARED((NP, 128), jnp.float32),
            pltpu.VMEM((CHUNK,), jnp.int32),
            pltpu.VMEM((NBC, K), jnp.int32),
            pltpu.VMEM((K, 128), jnp.float32),
            pltpu.VMEM((K, 128), jnp.float32),
            pltpu.VMEM((K, 128), jnp.float32),
            pltpu.VMEM((K, 128), jnp.float32),
            pltpu.SemaphoreType.DMA,
            pltpu.SemaphoreType.DMA,
        ],
    )
    def agg(feat_hbm, src_hbm, dst_hbm, zeros_hbm, out_hbm,
            accum, srcbuf, dstbuf, rows0, rows1, rows2, rows3, gsem, ssem):
        c = lax.axis_index("c")
        s = lax.axis_index("s")
        pltpu.async_copy(zeros_hbm, accum.at[pl.ds(s * RPT, RPT)], ssem)
        if split_edges:
            tile = c * NS + s
        else:
            tile = s

        def wait_gather(buf):
            pltpu.make_async_copy(
                feat_hbm.at[srcbuf.at[pl.ds(0, K)]], buf, gsem).wait()

        def wait_scatter():
            pltpu.make_async_copy(
                rows0, accum.at[dstbuf.at[0]], ssem).wait()

        def chunk_body(q, carry):
            pltpu.async_copy(src_hbm.at[pl.ds(tile * ept + q * CHUNK, CHUNK)],
                             srcbuf, gsem)
            pltpu.async_copy(dst_hbm.at[tile, q], dstbuf, gsem)
            pltpu.make_async_copy(
                src_hbm.at[pl.ds(0, CHUNK)], srcbuf, gsem).wait()
            pltpu.make_async_copy(dst_hbm.at[0, 0], dstbuf, gsem).wait()

            @pl.when(q == 0)
            def _():
                pltpu.make_async_copy(
                    zeros_hbm, accum.at[pl.ds(0, RPT)], ssem).wait()
                plsc.subcore_barrier()

            if not split_edges:
                def addoff(i, carry2):
                    srcbuf[pl.ds(i * 16, 16)] = (
                        srcbuf[pl.ds(i * 16, 16)] * 2 + c)
                    return carry2

                lax.fori_loop(0, CHUNK // 16, addoff, 0)

            bufs = (rows0, rows1, rows2, rows3)
            pltpu.async_copy(feat_hbm.at[srcbuf.at[pl.ds(0, K)]], rows0, gsem)
            pltpu.async_copy(feat_hbm.at[srcbuf.at[pl.ds(K, K)]], rows1, gsem)

            def step(j, carry2):
                def stage(cur, tgt):
                    wait_gather(cur)

                    @pl.when(j + 2 < NBC)
                    def _():
                        @pl.when(j >= 2)
                        def _():
                            wait_scatter()

                        pltpu.async_copy(
                            feat_hbm.at[srcbuf.at[pl.ds((j + 2) * K, K)]],
                            tgt, gsem)

                    pltpu.async_copy(cur, accum.at[dstbuf.at[j]], ssem,
                                     add=True)

                for p in range(4):
                    @pl.when(j % 4 == p)
                    def _(p=p):
                        stage(bufs[p], bufs[(p + 2) % 4])

                return carry2

            lax.fori_loop(0, NBC, step, 0)
            for _ in range(4):
                wait_scatter()
            return carry

        lax.fori_loop(0, nch, chunk_body, 0)
        plsc.subcore_barrier()
        pltpu.sync_copy(accum.at[pl.ds(s * RPT, RPT)],
                        out_hbm.at[c, pl.ds(s * RPT, RPT)])

    return agg


_agg_split = _make_agg(N, True)
_agg_feat = _make_agg(2 * N, False)


_BLK = 1000


def _prescale_call(degp, x):
    def body(degp_ref, x_ref, xs_ref, dis_ref):
        deg = degp_ref[0, :, 0:1] + degp_ref[1, :, 0:1] + 1.0
        d = lax.rsqrt(deg)
        dis_ref[...] = d
        xs_ref[...] = x_ref[...] * d

    return pl.pallas_call(
        body,
        grid=(N // _BLK,),
        in_specs=[
            pl.BlockSpec((NC, _BLK, 128), lambda i: (0, i, 0)),
            pl.BlockSpec((_BLK, 128), lambda i: (i, 0)),
        ],
        out_specs=[
            pl.BlockSpec((_BLK, 128), lambda i: (i, 0)),
            pl.BlockSpec((_BLK, 1), lambda i: (i, 0)),
        ],
        out_shape=[
            jax.ShapeDtypeStruct((N, 128), jnp.float32),
            jax.ShapeDtypeStruct((N, 1), jnp.float32),
        ],
    )(degp, x)


def _layer1_call(s1, x, dis, W1, b1, W2p):
    def body(s1_ref, x_ref, dis_ref, W1_ref, b1_ref, W2_ref, out_ref):
        d = dis_ref[...]
        agg = d * (s1_ref[0] + s1_ref[1]) + (d * d) * x_ref[...]
        h1 = jnp.maximum(
            jnp.dot(agg.astype(jnp.bfloat16), W1_ref[...],
                    preferred_element_type=jnp.float32)
            + b1_ref[...], 0.0)
        g = jnp.dot(h1.astype(jnp.bfloat16), W2_ref[...],
                    preferred_element_type=jnp.float32)
        out_ref[...] = d * g

    return pl.pallas_call(
        body,
        grid=(N // _BLK,),
        in_specs=[
            pl.BlockSpec((NC, _BLK, 128), lambda i: (0, i, 0)),
            pl.BlockSpec((_BLK, 128), lambda i: (i, 0)),
            pl.BlockSpec((_BLK, 1), lambda i: (i, 0)),
            pl.BlockSpec((128, 512), lambda i: (0, 0)),
            pl.BlockSpec((1, 512), lambda i: (0, 0)),
            pl.BlockSpec((512, 256), lambda i: (0, 0)),
        ],
        out_specs=pl.BlockSpec((_BLK, 256), lambda i: (i, 0)),
        out_shape=jax.ShapeDtypeStruct((N, 256), jnp.float32),
    )(s1, x, dis, W1.astype(jnp.bfloat16), b1, W2p.astype(jnp.bfloat16))


def _layer2_call(s2, gsc, dis, b2p):
    def body(s2_ref, gsc_ref, dis_ref, b2_ref, out_ref):
        d = dis_ref[...]
        b = jnp.where(pl.program_id(1) == 0, b2_ref[0:1, :], b2_ref[1:2, :])
        out_ref[...] = jnp.maximum(
            d * (s2_ref[0] + gsc_ref[...]) + b, 0.0)

    return pl.pallas_call(
        body,
        grid=(N // _BLK, 2),
        in_specs=[
            pl.BlockSpec((1, _BLK, 128), lambda i, j: (j, i, 0)),
            pl.BlockSpec((_BLK, 128), lambda i, j: (i, j)),
            pl.BlockSpec((_BLK, 1), lambda i, j: (i, 0)),
            pl.BlockSpec((2, 128), lambda i, j: (0, 0)),
        ],
        out_specs=pl.BlockSpec((_BLK, 128), lambda i, j: (i, j)),
        out_shape=jax.ShapeDtypeStruct((N, 256), jnp.float32),
    )(s2, gsc, dis, b2p)


def kernel(x, edge_index, W1, b1, W2, b2):
    src = edge_index[0]
    dst3s = edge_index[1].reshape(NC * NS, E // (NC * NS) // CHUNK, NBC, K)
    dst3f = edge_index[1].reshape(NS, E // NS // CHUNK, NBC, K)
    dst3d = edge_index[1].reshape(NC * NS, E // (NC * NS) // K, K)
    ones128 = jnp.ones((K, 128), jnp.float32)
    zeros128 = jnp.zeros((RPT, 128), jnp.float32)
    W2p = jnp.pad(W2, ((0, 0), (0, 256 - W2.shape[1])))
    b2p = jnp.pad(b2, (0, 256 - b2.shape[0])).reshape(2, 128)

    degp = _deg_kernel(dst3d, ones128, zeros128)
    xs, dis = _prescale_call(degp, x)
    s1 = _agg_split(xs, src, dst3s, zeros128)
    gsc = _layer1_call(s1, x, dis, W1, b1.reshape(1, 512), W2p)
    s2 = _agg_feat(gsc.reshape(2 * N, 128), src, dst3f, zeros128)
    o = _layer2_call(s2, gsc, dis, b2p)
    return o[:, :250]


# speedup vs baseline: 30.6959x; 1.0130x over previous
import functools

import jax
import jax.numpy as jnp
from jax import lax
from jax.experimental import pallas as pl
from jax.experimental.pallas import tpu as pltpu
from jax.experimental.pallas import tpu_sc as plsc

N = 10000
E = 320000
K = 80
NC, NS = 2, 16
NP = 10240
RPT = NP // NS
CHUNK = 2000
NBC = CHUNK // K

_sc_mesh = plsc.VectorSubcoreMesh(core_axis_name="c", subcore_axis_name="s")


@functools.partial(
    pl.kernel,
    out_type=jax.ShapeDtypeStruct((NC, NP, 128), jnp.float32),
    mesh=_sc_mesh,
    scratch_types=[
        pltpu.VMEM_SHARED((NP, 128), jnp.float32),
        pltpu.VMEM((E // (NC * NS) // K, K), jnp.int32),
        pltpu.VMEM((K, 128), jnp.float32),
        pltpu.SemaphoreType.DMA,
    ],
)
def _deg_kernel(dst_hbm, ones_hbm, zeros_hbm, out_hbm,
                accum, dstbuf, onesbuf, sem):
    c = lax.axis_index("c")
    s = lax.axis_index("s")
    nb = E // (NC * NS) // K
    pltpu.async_copy(zeros_hbm, accum.at[pl.ds(s * RPT, RPT)], sem)
    pltpu.async_copy(ones_hbm, onesbuf, sem)
    pltpu.async_copy(dst_hbm.at[c * NS + s], dstbuf, sem)
    pltpu.make_async_copy(zeros_hbm, accum.at[pl.ds(0, RPT)], sem).wait()
    pltpu.make_async_copy(ones_hbm, onesbuf, sem).wait()
    pltpu.make_async_copy(dst_hbm.at[0], dstbuf, sem).wait()
    plsc.subcore_barrier()

    def body(j, carry):
        pltpu.async_copy(onesbuf, accum.at[dstbuf.at[j]], sem, add=True)

        @pl.when(j >= 4)
        def _():
            pltpu.make_async_copy(onesbuf, accum.at[dstbuf.at[0]], sem).wait()

        return carry

    lax.fori_loop(0, nb, body, 0)
    for _ in range(4):
        pltpu.make_async_copy(onesbuf, accum.at[dstbuf.at[0]], sem).wait()
    plsc.subcore_barrier()
    pltpu.sync_copy(accum.at[pl.ds(s * RPT, RPT)],
                    out_hbm.at[c, pl.ds(s * RPT, RPT)])


def _make_agg(src_rows, split_edges):
    ept = E // (NC * NS) if split_edges else E // NS
    nch = ept // CHUNK

    @functools.partial(
        pl.kernel,
        out_type=jax.ShapeDtypeStruct((NC, NP, 128), jnp.float32),
        mesh=_sc_mesh,
        scratch_types=[
            pltpu.VMEM_SHARED((NP, 128), jnp.float32),
            pltpu.VMEM((CHUNK,), jnp.int32),
            pltpu.VMEM((NBC, K), jnp.int32),
            pltpu.VMEM((K, 128), jnp.float32),
            pltpu.VMEM((K, 128), jnp.float32),
            pltpu.VMEM((K, 128), jnp.float32),
            pltpu.VMEM((K, 128), jnp.float32),
            pltpu.SemaphoreType.DMA,
            pltpu.SemaphoreType.DMA,
        ],
    )
    def agg(feat_hbm, src_hbm, dst_hbm, zeros_hbm, out_hbm,
            accum, srcbuf, dstbuf, rows0, rows1, rows2, rows3, gsem, ssem):
        c = lax.axis_index("c")
        s = lax.axis_index("s")
        pltpu.async_copy(zeros_hbm, accum.at[pl.ds(s * RPT, RPT)], ssem)
        if split_edges:
            tile = c * NS + s
        else:
            tile = s

        def wait_gather(buf):
            pltpu.make_async_copy(
                feat_hbm.at[srcbuf.at[pl.ds(0, K)]], buf, gsem).wait()

        def wait_scatter():
            pltpu.make_async_copy(
                rows0, accum.at[dstbuf.at[0]], ssem).wait()

        def chunk_body(q, carry):
            pltpu.async_copy(src_hbm.at[pl.ds(tile * ept + q * CHUNK, CHUNK)],
                             srcbuf, gsem)
            pltpu.async_copy(dst_hbm.at[tile, q], dstbuf, gsem)
            pltpu.make_async_copy(
                src_hbm.at[pl.ds(0, CHUNK)], srcbuf, gsem).wait()
            pltpu.make_async_copy(dst_hbm.at[0, 0], dstbuf, gsem).wait()

            @pl.when(q == 0)
            def _():
                pltpu.make_async_copy(
                    zeros_hbm, accum.at[pl.ds(0, RPT)], ssem).wait()
                plsc.subcore_barrier()

            if not split_edges:
                def addoff(i, carry2):
                    srcbuf[pl.ds(i * 16, 16)] = (
                        srcbuf[pl.ds(i * 16, 16)] * 2 + c)
                    return carry2

                lax.fori_loop(0, CHUNK // 16, addoff, 0)

            bufs = (rows0, rows1, rows2, rows3)
            pltpu.async_copy(feat_hbm.at[srcbuf.at[pl.ds(0, K)]], rows0, gsem)
            pltpu.async_copy(feat_hbm.at[srcbuf.at[pl.ds(K, K)]], rows1, gsem)

            def step(j, carry2):
                def stage(cur, tgt):
                    wait_gather(cur)

                    @pl.when(j + 2 < NBC)
                    def _():
                        @pl.when(j >= 2)
                        def _():
                            wait_scatter()

                        pltpu.async_copy(
                            feat_hbm.at[srcbuf.at[pl.ds((j + 2) * K, K)]],
                            tgt, gsem)

                    pltpu.async_copy(cur, accum.at[dstbuf.at[j]], ssem,
                                     add=True)

                for p in range(4):
                    @pl.when(j % 4 == p)
                    def _(p=p):
                        stage(bufs[p], bufs[(p + 2) % 4])

                return carry2

            lax.fori_loop(0, NBC, step, 0)
            for _ in range(4):
                wait_scatter()
            return carry

        lax.fori_loop(0, nch, chunk_body, 0)
        plsc.subcore_barrier()
        pltpu.sync_copy(accum.at[pl.ds(s * RPT, RPT)],
                        out_hbm.at[c, pl.ds(s * RPT, RPT)])

    return agg


_agg_split = _make_agg(N, True)
_agg_feat = _make_agg(2 * N, False)


_BLK = 1000


def _prescale_call(degp, x):
    def body(degp_ref, x_ref, xs_ref, dis_ref):
        deg = degp_ref[0, :, 0:1] + degp_ref[1, :, 0:1] + 1.0
        d = lax.rsqrt(deg)
        dis_ref[...] = d
        xs_ref[...] = x_ref[...] * d

    return pl.pallas_call(
        body,
        grid=(N // _BLK,),
        in_specs=[
            pl.BlockSpec((NC, _BLK, 128), lambda i: (0, i, 0)),
            pl.BlockSpec((_BLK, 128), lambda i: (i, 0)),
        ],
        out_specs=[
            pl.BlockSpec((_BLK, 128), lambda i: (i, 0)),
            pl.BlockSpec((_BLK, 1), lambda i: (i, 0)),
        ],
        out_shape=[
            jax.ShapeDtypeStruct((N, 128), jnp.float32),
            jax.ShapeDtypeStruct((N, 1), jnp.float32),
        ],
    )(degp, x)


def _layer1_call(s1, x, dis, W1, b1, W2p):
    def body(s1_ref, x_ref, dis_ref, W1_ref, b1_ref, W2_ref, out_ref):
        d = dis_ref[...]
        agg = d * (s1_ref[0] + s1_ref[1]) + (d * d) * x_ref[...]
        h1 = jnp.maximum(
            jnp.dot(agg.astype(jnp.bfloat16), W1_ref[...],
                    preferred_element_type=jnp.float32)
            + b1_ref[...], 0.0)
        g = jnp.dot(h1.astype(jnp.bfloat16), W2_ref[...],
                    preferred_element_type=jnp.float32)
        out_ref[...] = d * g

    return pl.pallas_call(
        body,
        grid=(N // _BLK,),
        in_specs=[
            pl.BlockSpec((NC, _BLK, 128), lambda i: (0, i, 0)),
            pl.BlockSpec((_BLK, 128), lambda i: (i, 0)),
            pl.BlockSpec((_BLK, 1), lambda i: (i, 0)),
            pl.BlockSpec((128, 512), lambda i: (0, 0)),
            pl.BlockSpec((1, 512), lambda i: (0, 0)),
            pl.BlockSpec((512, 256), lambda i: (0, 0)),
        ],
        out_specs=pl.BlockSpec((_BLK, 256), lambda i: (i, 0)),
        out_shape=jax.ShapeDtypeStruct((N, 256), jnp.float32),
    )(s1, x, dis, W1.astype(jnp.bfloat16), b1, W2p.astype(jnp.bfloat16))


def _layer2_call(s2, gsc, dis, b2p):
    def body(s2_ref, gsc_ref, dis_ref, b2_ref, out_ref):
        d = dis_ref[...]
        s2cat = jnp.concatenate([s2_ref[0], s2_ref[1]], axis=1)
        val = jnp.maximum(d * (s2cat + gsc_ref[...]) + b2_ref[...], 0.0)
        out_ref[...] = val[:, :250]

    return pl.pallas_call(
        body,
        grid=(N // _BLK,),
        in_specs=[
            pl.BlockSpec((NC, _BLK, 128), lambda i: (0, i, 0)),
            pl.BlockSpec((_BLK, 256), lambda i: (i, 0)),
            pl.BlockSpec((_BLK, 1), lambda i: (i, 0)),
            pl.BlockSpec((1, 256), lambda i: (0, 0)),
        ],
        out_specs=pl.BlockSpec((_BLK, 250), lambda i: (i, 0)),
        out_shape=jax.ShapeDtypeStruct((N, 250), jnp.float32),
    )(s2, gsc, dis, b2p)


def kernel(x, edge_index, W1, b1, W2, b2):
    src = edge_index[0]
    dst3s = edge_index[1].reshape(NC * NS, E // (NC * NS) // CHUNK, NBC, K)
    dst3f = edge_index[1].reshape(NS, E // NS // CHUNK, NBC, K)
    dst3d = edge_index[1].reshape(NC * NS, E // (NC * NS) // K, K)
    ones128 = jnp.ones((K, 128), jnp.float32)
    zeros128 = jnp.zeros((RPT, 128), jnp.float32)
    W2p = jnp.pad(W2, ((0, 0), (0, 256 - W2.shape[1])))
    b2p = jnp.pad(b2, (0, 256 - b2.shape[0])).reshape(1, 256)

    degp = _deg_kernel(dst3d, ones128, zeros128)
    xs, dis = _prescale_call(degp, x)
    s1 = _agg_split(xs, src, dst3s, zeros128)
    gsc = _layer1_call(s1, x, dis, W1, b1.reshape(1, 512), W2p)
    s2 = _agg_feat(gsc.reshape(2 * N, 128), src, dst3f, zeros128)
    return _layer2_call(s2, gsc, dis, b2p)
